# baseline (speedup 1.0000x reference)
import functools

import jax
import jax.numpy as jnp
from jax import lax
from jax.experimental import pallas as pl
from jax.experimental.pallas import tpu as pltpu
from jax.experimental.pallas import tpu_sc as plsc

SEG = 16
EMB = 1024
LANES = 16
B_TOT = 4 * 8192
NC, NS = 2, 16
NW = NC * NS
BPW = B_TOT // NW
CH = 32
NCHUNK = BPW // CH

_SCALE = float(EMB) ** 0.5

_mesh = plsc.VectorSubcoreMesh(core_axis_name="c", subcore_axis_name="s")


@functools.partial(
    pl.kernel,
    out_type=jax.ShapeDtypeStruct((B_TOT, EMB), jnp.float32),
    mesh=_mesh,
    scratch_types=[
        pltpu.VMEM((BPW + LANES,), jnp.int32),
        pltpu.VMEM((SEG, EMB), jnp.float32),
        pltpu.VMEM((CH, EMB), jnp.float32),
        pltpu.VMEM((CH, EMB), jnp.float32),
        pltpu.SemaphoreType.DMA,
        pltpu.SemaphoreType.DMA,
    ],
)
def _emb_kernel(ids_hbm, w_hbm, out_hbm, idx_v, table_v, buf0, buf1, ws0, ws1):
    wid = lax.axis_index("s") * NC + lax.axis_index("c")
    base = wid * BPW

    pltpu.sync_copy(ids_hbm.at[pl.ds(base, BPW)], idx_v.at[pl.ds(0, BPW)])
    pltpu.sync_copy(w_hbm, table_v)

    def scale_row(r, carry):
        for j in range(EMB // LANES):
            table_v[r, pl.ds(j * LANES, LANES)] = (
                table_v[r, pl.ds(j * LANES, LANES)] * _SCALE
            )
        return carry

    lax.fori_loop(0, SEG, scale_row, 0)

    def build(k, buf):
        def row(r, carry):
            idv = idx_v[pl.ds(k * CH + r, LANES)][0]
            for j in range(EMB // LANES):
                buf[r, pl.ds(j * LANES, LANES)] = table_v[idv, pl.ds(j * LANES, LANES)]
            return carry

        lax.fori_loop(0, CH, row, 0)

    def w_start(k, buf, sem):
        pltpu.async_copy(buf, out_hbm.at[pl.ds(base + k * CH, CH)], sem)

    def w_wait(k, buf, sem):
        pltpu.make_async_copy(buf, out_hbm.at[pl.ds(base + k * CH, CH)], sem).wait()

    def do_chunk(k, buf, sem):
        @pl.when(k >= 2)
        def _():
            w_wait(k - 2, buf, sem)

        build(k, buf)
        w_start(k, buf, sem)

    def step(k, carry):
        @pl.when(k % 2 == 0)
        def _():
            do_chunk(k, buf0, ws0)

        @pl.when(k % 2 == 1)
        def _():
            do_chunk(k, buf1, ws1)

        return carry

    lax.fori_loop(0, NCHUNK, step, 0)

    w_wait(NCHUNK - 2, buf0, ws0)
    w_wait(NCHUNK - 1, buf1, ws1)


def kernel(segment_ids, weight):
    ids_flat = segment_ids.reshape(-1).astype(jnp.int32)
    out = _emb_kernel(ids_flat, weight)
    return out.reshape(segment_ids.shape + (EMB,))


# speedup vs baseline: 2.8197x; 2.8197x over previous
import functools

import jax
import jax.numpy as jnp
from jax import lax
from jax.experimental import pallas as pl
from jax.experimental.pallas import tpu as pltpu
from jax.experimental.pallas import tpu_sc as plsc

SEG = 16
EMB = 1024
LANES = 16
B_TOT = 4 * 8192
NC, NS = 2, 16
NW = NC * NS
BPW = B_TOT // NW
CH = 32
NCHUNK = BPW // CH

_SCALE = float(EMB) ** 0.5

_mesh = plsc.VectorSubcoreMesh(core_axis_name="c", subcore_axis_name="s")


@functools.partial(
    pl.kernel,
    out_type=jax.ShapeDtypeStruct((B_TOT, EMB), jnp.float32),
    mesh=_mesh,
    scratch_types=[
        pltpu.VMEM((BPW + LANES,), jnp.int32),
        pltpu.VMEM((SEG, EMB), jnp.float32),
        pltpu.VMEM((CH, EMB), jnp.float32),
        pltpu.VMEM((CH, EMB), jnp.float32),
        pltpu.SemaphoreType.DMA,
        pltpu.SemaphoreType.DMA,
    ],
)
def _emb_kernel(ids_hbm, w_hbm, out_hbm, idx_v, table_v, buf0, buf1, ws0, ws1):
    wid = lax.axis_index("s") * NC + lax.axis_index("c")
    base = wid * BPW

    pltpu.sync_copy(ids_hbm.at[pl.ds(base, BPW)], idx_v.at[pl.ds(0, BPW)])
    pltpu.sync_copy(w_hbm, table_v)

    def scale_row(r, carry):
        for j in range(EMB // LANES):
            table_v[r, pl.ds(j * LANES, LANES)] = (
                table_v[r, pl.ds(j * LANES, LANES)] * _SCALE
            )
        return carry

    lax.fori_loop(0, SEG, scale_row, 0)

    def build(k, buf):
        @plsc.parallel_loop(0, CH, 1, unroll=2)
        def _row(r):
            idv = idx_v[pl.ds(k * CH + r, LANES)][0]
            for g in range(4):
                vals = [
                    table_v[idv, pl.ds((g * 16 + j) * LANES, LANES)]
                    for j in range(16)
                ]
                for j in range(16):
                    buf[r, pl.ds((g * 16 + j) * LANES, LANES)] = vals[j]

    def w_start(k, buf, sem):
        pltpu.async_copy(buf, out_hbm.at[pl.ds(base + k * CH, CH)], sem)

    def w_wait(k, buf, sem):
        pltpu.make_async_copy(buf, out_hbm.at[pl.ds(base + k * CH, CH)], sem).wait()

    def do_chunk(k, buf, sem):
        @pl.when(k >= 2)
        def _():
            w_wait(k - 2, buf, sem)

        build(k, buf)
        w_start(k, buf, sem)

    def step(k, carry):
        @pl.when(k % 2 == 0)
        def _():
            do_chunk(k, buf0, ws0)

        @pl.when(k % 2 == 1)
        def _():
            do_chunk(k, buf1, ws1)

        return carry

    lax.fori_loop(0, NCHUNK, step, 0)

    w_wait(NCHUNK - 2, buf0, ws0)
    w_wait(NCHUNK - 1, buf1, ws1)


def kernel(segment_ids, weight):
    ids_flat = segment_ids.reshape(-1).astype(jnp.int32)
    out = _emb_kernel(ids_flat, weight)
    return out.reshape(segment_ids.shape + (EMB,))
